# baseline (speedup 1.0000x reference)
import functools

import jax
import jax.numpy as jnp
from jax import lax
from jax.experimental import pallas as pl
from jax.experimental.pallas import tpu as pltpu
from jax.experimental.pallas import tpu_sc as plsc

B = 16384
D = 32
H1 = 64
H2 = 32

_info = plsc.get_sparse_core_info()
_NC, _NS = _info.num_cores, _info.num_subcores
_NW = _NC * _NS
_BPW = B // _NW


_sc_mesh = plsc.VectorSubcoreMesh(core_axis_name="c", subcore_axis_name="s")


@functools.partial(
    pl.kernel,
    out_type=[
        jax.ShapeDtypeStruct((B, D), jnp.float32),
        jax.ShapeDtypeStruct((B, D), jnp.float32),
    ],
    mesh=_sc_mesh,
    compiler_params=pltpu.CompilerParams(use_tc_tiling_on_sc=False),
    scratch_types=[
        pltpu.VMEM((_BPW,), jnp.int32),
        pltpu.VMEM((_BPW, D), jnp.float32),
        pltpu.VMEM((_BPW,), jnp.int32),
        pltpu.VMEM((_BPW, D), jnp.float32),
        pltpu.SemaphoreType.DMA,
    ],
)
def _sc_gather(uemb_hbm, semb_hbm, uidx_hbm, sidx_hbm, uout_hbm, sout_hbm,
               uidx_v, urows_v, sidx_v, srows_v, sem):
    wid = lax.axis_index("s") * _NC + lax.axis_index("c")
    base = wid * _BPW
    pltpu.sync_copy(uidx_hbm.at[pl.ds(base, _BPW)], uidx_v)
    pltpu.sync_copy(sidx_hbm.at[pl.ds(base, _BPW)], sidx_v)
    cu = pltpu.async_copy(uemb_hbm.at[uidx_v], urows_v, sem)
    cs = pltpu.async_copy(semb_hbm.at[sidx_v], srows_v, sem)
    cu.wait()
    cs.wait()
    pltpu.sync_copy(urows_v, uout_hbm.at[pl.ds(base, _BPW)])
    pltpu.sync_copy(srows_v, sout_hbm.at[pl.ds(base, _BPW)])


def _mlp_body(u_ref, s_ref, wt_ref, w1u_ref, w1s_ref, w1c_ref, b1_ref,
              w2_ref, b2_ref, w3_ref, b3_ref, out_ref):
    x = (
        jnp.dot(u_ref[...], w1u_ref[...], preferred_element_type=jnp.float32)
        + jnp.dot(s_ref[...], w1s_ref[...], preferred_element_type=jnp.float32)
        + jnp.dot(wt_ref[...], w1c_ref[...], preferred_element_type=jnp.float32)
        + b1_ref[...]
    )
    h = jnp.maximum(x, 0.0)
    h = jnp.dot(h, w2_ref[...], preferred_element_type=jnp.float32) + b2_ref[...]
    h = jnp.maximum(h, 0.0)
    o = jnp.dot(h, w3_ref[...], preferred_element_type=jnp.float32) + b3_ref[...]
    out_ref[...] = jax.nn.sigmoid(o)


_MLP_BLK = 2048


def _mlp(u, s, wt, w1u, w1s, w1c, b1, w2, b2, w3, b3):
    grid = (B // _MLP_BLK,)
    return pl.pallas_call(
        _mlp_body,
        grid=grid,
        in_specs=[
            pl.BlockSpec((_MLP_BLK, D), lambda i: (i, 0)),
            pl.BlockSpec((_MLP_BLK, D), lambda i: (i, 0)),
            pl.BlockSpec((_MLP_BLK, 2), lambda i: (i, 0)),
            pl.BlockSpec((D, H1), lambda i: (0, 0)),
            pl.BlockSpec((D, H1), lambda i: (0, 0)),
            pl.BlockSpec((2, H1), lambda i: (0, 0)),
            pl.BlockSpec((1, H1), lambda i: (0, 0)),
            pl.BlockSpec((H1, H2), lambda i: (0, 0)),
            pl.BlockSpec((1, H2), lambda i: (0, 0)),
            pl.BlockSpec((H2, 1), lambda i: (0, 0)),
            pl.BlockSpec((1, 1), lambda i: (0, 0)),
        ],
        out_specs=pl.BlockSpec((_MLP_BLK, 1), lambda i: (i, 0)),
        out_shape=jax.ShapeDtypeStruct((B, 1), jnp.float32),
    )(u, s, wt, w1u, w1s, w1c, b1, w2, b2, w3, b3)


@jax.jit
def kernel(user, song, weather, time, user_emb, song_emb, W1, b1, W2, b2, W3, b3):
    uidx = user.astype(jnp.int32)
    sidx = song.astype(jnp.int32)
    u_rows, s_rows = _sc_gather(user_emb, song_emb, uidx, sidx)
    wt = jnp.stack([weather, time], axis=1)
    w1u = W1[:D]
    w1s = W1[D:2 * D]
    w1c = W1[2 * D:]
    out = _mlp(u_rows, s_rows, wt, w1u, w1s, w1c, b1[None, :],
               W2, b2[None, :], W3, b3[None, :])
    return jnp.squeeze(out, axis=-1)
